# baseline (speedup 1.0000x reference)
import functools

import jax
import jax.numpy as jnp
from jax import lax
from jax.experimental import pallas as pl
from jax.experimental.pallas import tpu as pltpu
from jax.experimental.pallas import tpu_sc as plsc

N = 10000
NP = 10112
IN_F = 128
H1 = 4
C1 = 32
D1 = 128
C2 = 32
NG = 64
EPS = 1e-5
NW = 32
CH = 128
EP = 331776
PW = EP // NW
NCHUNK = PW // CH
PW2 = EP // 16
NCHUNK2 = PW2 // CH
EB = 4096
NEB = EP // EB
RB = 1264
NRB = NP // RB
DW = 128
KG = 3
SPAN = NP // 16
WBS = [128, 128, 128, 128, 120]

_f32 = jnp.float32


def _sc_gather2(xl, xr, src, dst, ebase, nckw):
    D = xl.shape[1]
    mesh = plsc.VectorSubcoreMesh(core_axis_name="c", subcore_axis_name="s")
    pwl = nckw * CH
    eh = pwl * NW
    ngrp = nckw // KG
    ntail = nckw - ngrp * KG

    @functools.partial(
        pl.kernel,
        out_type=(jax.ShapeDtypeStruct((eh, D), _f32),
                  jax.ShapeDtypeStruct((eh, D), _f32)),
        mesh=mesh,
        scratch_types=[
            pltpu.VMEM((pwl,), jnp.int32),
            pltpu.VMEM((pwl,), jnp.int32),
        ] + [pltpu.VMEM((CH, D), _f32)] * (2 * KG) + [
            pltpu.SemaphoreType.DMA,
            pltpu.SemaphoreType.DMA,
        ],
    )
    def k(xl_hbm, xr_hbm, src_hbm, dst_hbm, outl_hbm, outr_hbm,
          si_v, di_v, *bufs_sems):
        rl = bufs_sems[0:KG]
        rr = bufs_sems[KG:2 * KG]
        semg, semo = bufs_sems[2 * KG], bufs_sems[2 * KG + 1]
        wid = lax.axis_index("s") * 2 + lax.axis_index("c")
        base = wid * pwl
        pltpu.sync_copy(src_hbm.at[pl.ds(ebase * CH + base, pwl)], si_v)
        pltpu.sync_copy(dst_hbm.at[pl.ds(ebase * CH + base, pwl)], di_v)

        def do_grp(i0, nb):
            ds = []
            for b in range(nb):
                lo = (i0 + b) * CH
                ds.append(pltpu.async_copy(
                    xl_hbm.at[si_v.at[pl.ds(lo, CH)]], rl[b], semg))
                ds.append(pltpu.async_copy(
                    xr_hbm.at[di_v.at[pl.ds(lo, CH)]], rr[b], semg))
            for d in ds:
                d.wait()
            ds = []
            for b in range(nb):
                off = base + (i0 + b) * CH
                ds.append(pltpu.async_copy(
                    rl[b], outl_hbm.at[pl.ds(off, CH)], semo))
                ds.append(pltpu.async_copy(
                    rr[b], outr_hbm.at[pl.ds(off, CH)], semo))
            for d in ds:
                d.wait()

        def grp(g, c):
            do_grp(g * KG, KG)
            return c

        lax.fori_loop(0, ngrp, grp, 0)
        if ntail:
            do_grp(ngrp * KG, ntail)

    return k(xl, xr, src, dst)


def _zero_vmem(buf):
    nlanes = DW // 16

    def zb_body(i, c):
        j = i // nlanes
        kk = i % nlanes
        buf[j, pl.ds(kk * 16, 16)] = jnp.zeros((16,), _f32)
        return c

    lax.fori_loop(0, CH * nlanes, zb_body, 0)


def _sc_scatter(staged, dst, headsplit, ebase, nch):
    mesh = plsc.VectorSubcoreMesh(core_axis_name="c", subcore_axis_name="s")
    nckw = nch // 16 if headsplit else nch // 32
    pw = nckw * CH
    ngrp = nckw // KG
    ntail = nckw - ngrp * KG

    @functools.partial(
        pl.kernel,
        out_type=jax.ShapeDtypeStruct((2, NP, DW), _f32),
        mesh=mesh,
        scratch_types=[pltpu.VMEM((CH,), jnp.int32)] * KG
        + [pltpu.VMEM((CH, DW), _f32)] * KG + [
            pltpu.VMEM_SHARED((NP, DW), _f32),
            pltpu.SemaphoreType.DMA,
            pltpu.SemaphoreType.DMA,
        ],
    )
    def k(staged_hbm, dst_hbm, out_hbm, *rest):
        idx = rest[0:KG]
        rows = rest[KG:2 * KG]
        acc_sh, semi, semo = rest[2 * KG], rest[2 * KG + 1], rest[2 * KG + 2]
        cid = lax.axis_index("c")
        sid = lax.axis_index("s")
        if headsplit:
            base = sid * pw
        else:
            base = (sid * 2 + cid) * pw

        _zero_vmem(rows[0])
        zds = []
        r0 = sid * SPAN
        for sz in WBS:
            zds.append(pltpu.async_copy(
                rows[0].at[pl.ds(0, sz)], acc_sh.at[pl.ds(r0, sz)], semo))
            r0 += sz
        for d in zds:
            d.wait()
        plsc.subcore_barrier()

        def do_grp(i0, nb):
            ds = []
            for b in range(nb):
                off = base + (i0 + b) * CH
                ds.append(pltpu.async_copy(
                    dst_hbm.at[pl.ds(ebase * CH + off, CH)], idx[b], semi))
                if headsplit:
                    sref = staged_hbm.at[cid, pl.ds(off, CH)]
                else:
                    sref = staged_hbm.at[pl.ds(off, CH)]
                ds.append(pltpu.async_copy(sref, rows[b], semi))
            for d in ds:
                d.wait()
            ds = []
            for b in range(nb):
                ds.append(pltpu.async_copy(
                    rows[b], acc_sh.at[idx[b]], semo, add=True))
            for d in ds:
                d.wait()

        def grp(g, c):
            do_grp(g * KG, KG)
            return c

        lax.fori_loop(0, ngrp, grp, 0)
        if ntail:
            do_grp(ngrp * KG, ntail)
        plsc.subcore_barrier()

        outds = []
        r0 = sid * SPAN
        for t, sz in enumerate(WBS):
            if t >= KG:
                outds[t - KG].wait()
            pltpu.sync_copy(acc_sh.at[pl.ds(r0, sz)],
                            rows[t % KG].at[pl.ds(0, sz)])
            outds.append(pltpu.async_copy(
                rows[t % KG].at[pl.ds(0, sz)],
                out_hbm.at[cid, pl.ds(r0, sz)], semo))
            r0 += sz
        for d in outds[max(0, len(WBS) - KG):]:
            d.wait()

    return k(staged, dst)


def _mm2_body(x_ref, wl_ref, wr_ref, aabs_ref, xl_ref, xr_ref,
              bl_ref, br_ref):
    xb = x_ref[...]
    xl = jnp.dot(xb, wl_ref[...], preferred_element_type=_f32)
    xr = jnp.dot(xb, wr_ref[...], preferred_element_type=_f32)
    xl_ref[...] = xl
    xr_ref[...] = xr
    aabs = aabs_ref[...]
    blb = jnp.max(jnp.dot(jnp.abs(xl), aabs, preferred_element_type=_f32),
                  axis=0, keepdims=True)
    brb = jnp.max(jnp.dot(jnp.abs(xr), aabs, preferred_element_type=_f32),
                  axis=0, keepdims=True)

    @pl.when(pl.program_id(0) == 0)
    def _():
        bl_ref[...] = jnp.zeros(bl_ref.shape, _f32)
        br_ref[...] = jnp.zeros(br_ref.shape, _f32)

    bl_ref[...] = jnp.maximum(bl_ref[...], blb)
    br_ref[...] = jnp.maximum(br_ref[...], brb)


def _tc_mm2(x, wl, wr, aabs):
    D_in, D_out = wl.shape
    HH = aabs.shape[1]
    return pl.pallas_call(
        _mm2_body,
        grid=(NRB,),
        in_specs=[
            pl.BlockSpec((RB, D_in), lambda i: (i, 0)),
            pl.BlockSpec((D_in, D_out), lambda i: (0, 0)),
            pl.BlockSpec((D_in, D_out), lambda i: (0, 0)),
            pl.BlockSpec((D_out, HH), lambda i: (0, 0)),
        ],
        out_specs=[
            pl.BlockSpec((RB, D_out), lambda i: (i, 0)),
            pl.BlockSpec((RB, D_out), lambda i: (i, 0)),
            pl.BlockSpec((8, HH), lambda i: (0, 0)),
            pl.BlockSpec((8, HH), lambda i: (0, 0)),
        ],
        out_shape=[
            jax.ShapeDtypeStruct((NP, D_out), _f32),
            jax.ShapeDtypeStruct((NP, D_out), _f32),
            jax.ShapeDtypeStruct((8, HH), _f32),
            jax.ShapeDtypeStruct((8, HH), _f32),
        ],
    )(x, wl, wr, aabs)


def _edge1_body(xls_ref, xrd_ref, a_ref, bl_ref, br_ref, out_ref):
    bound = jnp.max(bl_ref[...]) + jnp.max(br_ref[...])
    z = xls_ref[...] + xrd_ref[...]
    m = jnp.maximum(z, 0.2 * z)
    e = jnp.dot(m, a_ref[...], preferred_element_type=_f32)
    w = jnp.exp(e - bound)
    xls = xls_ref[...]
    zpad = jnp.zeros((EB, DW - 2 * C1 - 2), _f32)
    out_ref[0] = jnp.concatenate(
        [w[:, 0:1] * xls[:, 0:C1], w[:, 1:2] * xls[:, C1:2 * C1],
         w[:, 0:2], zpad], axis=1)
    out_ref[1] = jnp.concatenate(
        [w[:, 2:3] * xls[:, 2 * C1:3 * C1], w[:, 3:4] * xls[:, 3 * C1:4 * C1],
         w[:, 2:4], zpad], axis=1)


def _tc_edge1(xls, xrd, a_mat, bl, br):
    eh = xls.shape[0]
    return pl.pallas_call(
        _edge1_body,
        grid=(eh // EB,),
        in_specs=[
            pl.BlockSpec((EB, D1), lambda i: (i, 0)),
            pl.BlockSpec((EB, D1), lambda i: (i, 0)),
            pl.BlockSpec((D1, H1), lambda i: (0, 0)),
            pl.BlockSpec((8, H1), lambda i: (0, 0)),
            pl.BlockSpec((8, H1), lambda i: (0, 0)),
        ],
        out_specs=pl.BlockSpec((2, EB, DW), lambda i: (0, i, 0)),
        out_shape=jax.ShapeDtypeStruct((2, eh, DW), _f32),
    )(xls, xrd, a_mat, bl, br)


def _edge2_body(xls_ref, xrd_ref, a_ref, bl_ref, br_ref, out_ref):
    bound = jnp.max(bl_ref[...]) + jnp.max(br_ref[...])
    z = xls_ref[...] + xrd_ref[...]
    m = jnp.maximum(z, 0.2 * z)
    e = jnp.dot(m, a_ref[...], preferred_element_type=_f32)
    w = jnp.exp(e - bound)
    out_ref[...] = jnp.concatenate(
        [w * xls_ref[:, :C2], w, jnp.zeros((EB, DW - C2 - 1), _f32)],
        axis=1)


def _tc_edge2(xls, xrd, a_mat, bl, br):
    eh = xls.shape[0]
    return pl.pallas_call(
        _edge2_body,
        grid=(eh // EB,),
        in_specs=[
            pl.BlockSpec((EB, DW), lambda i: (i, 0)),
            pl.BlockSpec((EB, DW), lambda i: (i, 0)),
            pl.BlockSpec((DW, 1), lambda i: (0, 0)),
            pl.BlockSpec((8, 1), lambda i: (0, 0)),
            pl.BlockSpec((8, 1), lambda i: (0, 0)),
        ],
        out_specs=pl.BlockSpec((EB, DW), lambda i: (i, 0)),
        out_shape=jax.ShapeDtypeStruct((eh, DW), _f32),
    )(xls, xrd, a_mat, bl, br)


def _combine1_body(*refs):
    nsh = (len(refs) - 5) // 2
    p0 = refs[0][...]
    p1 = refs[nsh][...]
    for s in range(1, nsh):
        p0 = p0 + refs[s][...]
        p1 = p1 + refs[nsh + s][...]
    st_ref, b_ref, h_ref, ps_ref, pq_ref = refs[2 * nsh:]
    num = jnp.concatenate([p0[:, 0:2 * C1], p1[:, 0:2 * C1]], axis=1)
    den4 = jnp.concatenate([p0[:, 2 * C1:2 * C1 + 2],
                            p1[:, 2 * C1:2 * C1 + 2]], axis=1)
    den = jnp.dot(den4, st_ref[...], preferred_element_type=_f32) + 1e-16
    h = jnp.maximum(num / den + b_ref[...], 0.0)
    rows = (pl.program_id(0) * RB
            + lax.broadcasted_iota(jnp.int32, (RB, 1), 0))
    h = jnp.where(rows < N, h, 0.0)
    h_ref[...] = h

    @pl.when(pl.program_id(0) == 0)
    def _():
        ps_ref[...] = jnp.zeros(ps_ref.shape, _f32)
        pq_ref[...] = jnp.zeros(pq_ref.shape, _f32)

    ps_ref[...] = ps_ref[...] + jnp.sum(h, axis=0, keepdims=True)
    pq_ref[...] = pq_ref[...] + jnp.sum(h * h, axis=0, keepdims=True)


def _tc_combine1(p0s, p1s, st_mat, b1):
    nsh = len(p0s)
    return pl.pallas_call(
        _combine1_body,
        grid=(NRB,),
        in_specs=[pl.BlockSpec((RB, DW), lambda i: (i, 0))] * (2 * nsh) + [
            pl.BlockSpec((H1, D1), lambda i: (0, 0)),
            pl.BlockSpec((1, D1), lambda i: (0, 0)),
        ],
        out_specs=[
            pl.BlockSpec((RB, D1), lambda i: (i, 0)),
            pl.BlockSpec((8, D1), lambda i: (0, 0)),
            pl.BlockSpec((8, D1), lambda i: (0, 0)),
        ],
        out_shape=[
            jax.ShapeDtypeStruct((NP, D1), _f32),
            jax.ShapeDtypeStruct((8, D1), _f32),
            jax.ShapeDtypeStruct((8, D1), _f32),
        ],
    )(*p0s, *p1s, st_mat, b1)


def _bn_mm2_body(h_ref, ps_ref, pq_ref, g_ref, bt_ref, wl_ref, wr_ref,
                 aabs_ref, xl_ref, xr_ref, bl_ref, br_ref):
    mu = ps_ref[0:1, :] / N
    var = pq_ref[0:1, :] / N - mu * mu
    hn = (h_ref[...] - mu) / jnp.sqrt(var + EPS) * g_ref[...] + bt_ref[...]
    rows = lax.broadcasted_iota(jnp.int32, (NP, 1), 0)
    hn = jnp.where(rows < N, hn, 0.0)
    xl = jnp.dot(hn, wl_ref[...], preferred_element_type=_f32)
    xr = jnp.dot(hn, wr_ref[...], preferred_element_type=_f32)
    xl_ref[...] = xl
    xr_ref[...] = xr
    aabs = aabs_ref[...]
    zero8 = jnp.zeros((8, 1), _f32)
    bl_ref[...] = zero8 + jnp.max(
        jnp.dot(jnp.abs(xl), aabs, preferred_element_type=_f32))
    br_ref[...] = zero8 + jnp.max(
        jnp.dot(jnp.abs(xr), aabs, preferred_element_type=_f32))


def _tc_bn_mm2(h, ps, pq, g, bt, wl, wr, aabs):
    return pl.pallas_call(
        _bn_mm2_body,
        out_shape=[
            jax.ShapeDtypeStruct((NP, DW), _f32),
            jax.ShapeDtypeStruct((NP, DW), _f32),
            jax.ShapeDtypeStruct((8, 1), _f32),
            jax.ShapeDtypeStruct((8, 1), _f32),
        ],
    )(h, ps, pq, g, bt, wl, wr, aabs)


def _final_body(*refs):
    nparts = len(refs) - 7
    b_ref, g_ref, bt_ref, batch_ref, wlin_ref, blin_ref, out_ref = \
        refs[nparts:]
    acc = refs[0][...]
    for s in range(1, nparts):
        acc = acc + refs[s][...]
    den = acc[:, C2:C2 + 1] + 1e-16
    h = jnp.maximum(acc[:, :C2] / den + b_ref[...], 0.0)
    rows = lax.broadcasted_iota(jnp.int32, (NP, 1), 0)
    h = jnp.where(rows < N, h, 0.0)
    mu = jnp.sum(h, axis=0, keepdims=True) / N
    var = jnp.sum(h * h, axis=0, keepdims=True) / N - mu * mu
    hn = (h - mu) / jnp.sqrt(var + EPS) * g_ref[...] + bt_ref[...]
    hn = jnp.where(rows < N, hn, 0.0)
    batch = batch_ref[...]

    def pool_body(g, pool):
        bb = batch == g
        s = jnp.sum(jnp.where(bb, hn, 0.0), axis=0, keepdims=True)
        cnt = jnp.sum(jnp.where(bb, 1.0, 0.0))
        mx = jnp.max(jnp.where(bb, hn, -jnp.inf), axis=0, keepdims=True)
        mean = s / jnp.maximum(cnt, 1.0)
        mx = jnp.where(cnt > 0, mx, 0.0)
        row = jnp.concatenate([s, mean, mx], axis=1)
        sel = (lax.broadcasted_iota(jnp.int32, (NG, 1), 0) == g
               ).astype(_f32)
        return pool + sel * row

    pool = lax.fori_loop(0, NG, pool_body, jnp.zeros((NG, 3 * C2), _f32))
    out_ref[...] = (jnp.dot(pool, wlin_ref[...], preferred_element_type=_f32)
                    + blin_ref[...])


def _tc_final(parts, b2, g2, bt2, batch_p, wlin, blin):
    return pl.pallas_call(
        _final_body,
        out_shape=jax.ShapeDtypeStruct((NG, 16), _f32),
    )(*parts, b2, g2, bt2, batch_p, wlin, blin)


def kernel(x, edge_index, batch, Wl1, Wr1, att1, b1, g1, bt1,
           Wl2, Wr2, att2, b2, g2, bt2, Wlin, blin):
    idt = edge_index.dtype
    sl = jnp.arange(N, dtype=idt)
    npad = EP - (edge_index.shape[1] + N)
    src = jnp.concatenate(
        [edge_index[0], sl, jnp.full((npad,), N, idt)]).astype(jnp.int32)
    dst = jnp.concatenate(
        [edge_index[1], sl, jnp.full((npad,), N, idt)]).astype(jnp.int32)
    x_p = jnp.pad(x, ((0, NP - N), (0, 0)))
    batch_p = jnp.pad(batch, (0, NP - N), constant_values=NG)
    batch_p = batch_p.reshape(NP, 1).astype(jnp.int32)

    ch = jnp.arange(D1)
    A1 = jnp.zeros((D1, H1), _f32).at[ch, ch // C1].set(
        att1[ch // C1, ch % C1])
    ST1 = (ch[None, :] // C1 == jnp.arange(H1)[:, None]).astype(_f32)
    A2 = jnp.pad(att2.reshape(C2, 1), ((0, DW - C2), (0, 0)))
    Wl2p = jnp.pad(Wl2, ((0, 0), (0, DW - C2)))
    Wr2p = jnp.pad(Wr2, ((0, 0), (0, DW - C2)))

    NSH = 3
    CSH = NCHUNK * NW // NSH
    bases = [s * CSH for s in range(NSH)]
    xl1, xr1, bl1, br1 = _tc_mm2(x_p, Wl1, Wr1, jnp.abs(A1))
    g1s = [_sc_gather2(xl1, xr1, src, dst, b_, CSH // NW) for b_ in bases]
    st1s = [_tc_edge1(gl, gr, A1, bl1, br1) for gl, gr in g1s]
    p1s = [_sc_scatter(st, dst, True, b_, CSH)
           for st, b_ in zip(st1s, bases)]
    h1, ps1, pq1 = _tc_combine1([p[0] for p in p1s], [p[1] for p in p1s],
                                ST1, b1.reshape(1, D1))
    xl2, xr2, bl2, br2 = _tc_bn_mm2(h1, ps1, pq1, g1.reshape(1, D1),
                                    bt1.reshape(1, D1), Wl2p, Wr2p,
                                    jnp.abs(A2))

    g2s = [_sc_gather2(xl2, xr2, src, dst, b_, CSH // NW) for b_ in bases]
    st2s = [_tc_edge2(gl, gr, A2, bl2, br2) for gl, gr in g2s]
    p2s = [_sc_scatter(st, dst, False, b_, CSH)
           for st, b_ in zip(st2s, bases)]

    return _tc_final([p[0] for p in p2s] + [p[1] for p in p2s],
                     b2.reshape(1, C2), g2.reshape(1, C2),
                     bt2.reshape(1, C2), batch_p, Wlin,
                     blin.reshape(1, 16))


# speedup vs baseline: 1.1063x; 1.1063x over previous
import functools

import jax
import jax.numpy as jnp
from jax import lax
from jax.experimental import pallas as pl
from jax.experimental.pallas import tpu as pltpu
from jax.experimental.pallas import tpu_sc as plsc

N = 10000
NP = 10112
IN_F = 128
H1 = 4
C1 = 32
D1 = 128
C2 = 32
NG = 64
EPS = 1e-5
NW = 32
CH = 128
EP = 331776
PW = EP // NW
NCHUNK = PW // CH
PW2 = EP // 16
NCHUNK2 = PW2 // CH
EB = 4096
NEB = EP // EB
RB = 1264
NRB = NP // RB
DW = 128
KG = 3
SPAN = NP // 16
WBS = [128, 128, 128, 128, 120]

_f32 = jnp.float32


def _sc_gather2(xl, xr, src, dst, ebase, nckw):
    D = xl.shape[1]
    mesh = plsc.VectorSubcoreMesh(core_axis_name="c", subcore_axis_name="s")
    pwl = nckw * CH
    eh = pwl * NW
    ngrp = nckw // KG
    ntail = nckw - ngrp * KG

    @functools.partial(
        pl.kernel,
        out_type=(jax.ShapeDtypeStruct((eh, D), _f32),
                  jax.ShapeDtypeStruct((eh, D), _f32)),
        mesh=mesh,
        scratch_types=[
            pltpu.VMEM((pwl,), jnp.int32),
            pltpu.VMEM((pwl,), jnp.int32),
        ] + [pltpu.VMEM((CH, D), _f32)] * (2 * KG) + [
            pltpu.SemaphoreType.DMA,
            pltpu.SemaphoreType.DMA,
        ],
    )
    def k(xl_hbm, xr_hbm, src_hbm, dst_hbm, outl_hbm, outr_hbm,
          si_v, di_v, *bufs_sems):
        rl = bufs_sems[0:KG]
        rr = bufs_sems[KG:2 * KG]
        semg, semo = bufs_sems[2 * KG], bufs_sems[2 * KG + 1]
        wid = lax.axis_index("s") * 2 + lax.axis_index("c")
        base = wid * pwl
        pltpu.sync_copy(src_hbm.at[pl.ds(ebase * CH + base, pwl)], si_v)
        pltpu.sync_copy(dst_hbm.at[pl.ds(ebase * CH + base, pwl)], di_v)

        gds = {}
        ods = {}

        def fire_out(i):
            b = i % KG
            gds[i][0].wait()
            gds[i][1].wait()
            off = base + i * CH
            ods[i] = (
                pltpu.async_copy(rl[b], outl_hbm.at[pl.ds(off, CH)], semo),
                pltpu.async_copy(rr[b], outr_hbm.at[pl.ds(off, CH)], semo),
            )

        for i in range(nckw):
            b = i % KG
            if i >= KG:
                ods[i - KG][0].wait()
                ods[i - KG][1].wait()
            lo = i * CH
            gds[i] = (
                pltpu.async_copy(
                    xl_hbm.at[si_v.at[pl.ds(lo, CH)]], rl[b], semg),
                pltpu.async_copy(
                    xr_hbm.at[di_v.at[pl.ds(lo, CH)]], rr[b], semg),
            )
            if i >= 1:
                fire_out(i - 1)
        fire_out(nckw - 1)
        for i in range(max(0, nckw - KG), nckw):
            ods[i][0].wait()
            ods[i][1].wait()

    return k(xl, xr, src, dst)


def _zero_vmem(buf):
    nlanes = DW // 16

    def zb_body(i, c):
        j = i // nlanes
        kk = i % nlanes
        buf[j, pl.ds(kk * 16, 16)] = jnp.zeros((16,), _f32)
        return c

    lax.fori_loop(0, CH * nlanes, zb_body, 0)


def _sc_scatter(staged, dst, headsplit, ebase, nch):
    mesh = plsc.VectorSubcoreMesh(core_axis_name="c", subcore_axis_name="s")
    nckw = nch // 16 if headsplit else nch // 32
    pw = nckw * CH
    ngrp = nckw // KG
    ntail = nckw - ngrp * KG

    @functools.partial(
        pl.kernel,
        out_type=jax.ShapeDtypeStruct((2, NP, DW), _f32),
        mesh=mesh,
        scratch_types=[pltpu.VMEM((CH,), jnp.int32)] * KG
        + [pltpu.VMEM((CH, DW), _f32)] * KG + [
            pltpu.VMEM_SHARED((NP, DW), _f32),
            pltpu.SemaphoreType.DMA,
            pltpu.SemaphoreType.DMA,
        ],
    )
    def k(staged_hbm, dst_hbm, out_hbm, *rest):
        idx = rest[0:KG]
        rows = rest[KG:2 * KG]
        acc_sh, semi, semo = rest[2 * KG], rest[2 * KG + 1], rest[2 * KG + 2]
        cid = lax.axis_index("c")
        sid = lax.axis_index("s")
        if headsplit:
            base = sid * pw
        else:
            base = (sid * 2 + cid) * pw

        _zero_vmem(rows[0])
        zds = []
        r0 = sid * SPAN
        for sz in WBS:
            zds.append(pltpu.async_copy(
                rows[0].at[pl.ds(0, sz)], acc_sh.at[pl.ds(r0, sz)], semo))
            r0 += sz
        for d in zds:
            d.wait()
        plsc.subcore_barrier()

        lds = {}
        sds = {}

        def fire_scatter(i):
            b = i % KG
            lds[i][0].wait()
            lds[i][1].wait()
            sds[i] = pltpu.async_copy(rows[b], acc_sh.at[idx[b]], semo,
                                      add=True)

        for i in range(nckw):
            b = i % KG
            if i >= KG:
                sds[i - KG].wait()
            off = base + i * CH
            if headsplit:
                sref = staged_hbm.at[cid, pl.ds(off, CH)]
            else:
                sref = staged_hbm.at[pl.ds(off, CH)]
            lds[i] = (
                pltpu.async_copy(
                    dst_hbm.at[pl.ds(ebase * CH + off, CH)], idx[b], semi),
                pltpu.async_copy(sref, rows[b], semi),
            )
            if i >= 1:
                fire_scatter(i - 1)
        fire_scatter(nckw - 1)
        for i in range(max(0, nckw - KG), nckw):
            sds[i].wait()
        plsc.subcore_barrier()

        outds = []
        r0 = sid * SPAN
        for t, sz in enumerate(WBS):
            if t >= KG:
                outds[t - KG].wait()
            pltpu.sync_copy(acc_sh.at[pl.ds(r0, sz)],
                            rows[t % KG].at[pl.ds(0, sz)])
            outds.append(pltpu.async_copy(
                rows[t % KG].at[pl.ds(0, sz)],
                out_hbm.at[cid, pl.ds(r0, sz)], semo))
            r0 += sz
        for d in outds[max(0, len(WBS) - KG):]:
            d.wait()

    return k(staged, dst)


def _mm2_body(x_ref, wl_ref, wr_ref, aabs_ref, xl_ref, xr_ref,
              bl_ref, br_ref):
    xb = x_ref[...]
    xl = jnp.dot(xb, wl_ref[...], preferred_element_type=_f32)
    xr = jnp.dot(xb, wr_ref[...], preferred_element_type=_f32)
    xl_ref[...] = xl
    xr_ref[...] = xr
    aabs = aabs_ref[...]
    blb = jnp.max(jnp.dot(jnp.abs(xl), aabs, preferred_element_type=_f32),
                  axis=0, keepdims=True)
    brb = jnp.max(jnp.dot(jnp.abs(xr), aabs, preferred_element_type=_f32),
                  axis=0, keepdims=True)

    @pl.when(pl.program_id(0) == 0)
    def _():
        bl_ref[...] = jnp.zeros(bl_ref.shape, _f32)
        br_ref[...] = jnp.zeros(br_ref.shape, _f32)

    bl_ref[...] = jnp.maximum(bl_ref[...], blb)
    br_ref[...] = jnp.maximum(br_ref[...], brb)


def _tc_mm2(x, wl, wr, aabs):
    D_in, D_out = wl.shape
    HH = aabs.shape[1]
    return pl.pallas_call(
        _mm2_body,
        grid=(NRB,),
        in_specs=[
            pl.BlockSpec((RB, D_in), lambda i: (i, 0)),
            pl.BlockSpec((D_in, D_out), lambda i: (0, 0)),
            pl.BlockSpec((D_in, D_out), lambda i: (0, 0)),
            pl.BlockSpec((D_out, HH), lambda i: (0, 0)),
        ],
        out_specs=[
            pl.BlockSpec((RB, D_out), lambda i: (i, 0)),
            pl.BlockSpec((RB, D_out), lambda i: (i, 0)),
            pl.BlockSpec((8, HH), lambda i: (0, 0)),
            pl.BlockSpec((8, HH), lambda i: (0, 0)),
        ],
        out_shape=[
            jax.ShapeDtypeStruct((NP, D_out), _f32),
            jax.ShapeDtypeStruct((NP, D_out), _f32),
            jax.ShapeDtypeStruct((8, HH), _f32),
            jax.ShapeDtypeStruct((8, HH), _f32),
        ],
    )(x, wl, wr, aabs)


def _edge1_body(xls_ref, xrd_ref, a_ref, bl_ref, br_ref, out_ref):
    bound = jnp.max(bl_ref[...]) + jnp.max(br_ref[...])
    z = xls_ref[...] + xrd_ref[...]
    m = jnp.maximum(z, 0.2 * z)
    e = jnp.dot(m, a_ref[...], preferred_element_type=_f32)
    w = jnp.exp(e - bound)
    xls = xls_ref[...]
    zpad = jnp.zeros((EB, DW - 2 * C1 - 2), _f32)
    out_ref[0] = jnp.concatenate(
        [w[:, 0:1] * xls[:, 0:C1], w[:, 1:2] * xls[:, C1:2 * C1],
         w[:, 0:2], zpad], axis=1)
    out_ref[1] = jnp.concatenate(
        [w[:, 2:3] * xls[:, 2 * C1:3 * C1], w[:, 3:4] * xls[:, 3 * C1:4 * C1],
         w[:, 2:4], zpad], axis=1)


def _tc_edge1(xls, xrd, a_mat, bl, br):
    eh = xls.shape[0]
    return pl.pallas_call(
        _edge1_body,
        grid=(eh // EB,),
        in_specs=[
            pl.BlockSpec((EB, D1), lambda i: (i, 0)),
            pl.BlockSpec((EB, D1), lambda i: (i, 0)),
            pl.BlockSpec((D1, H1), lambda i: (0, 0)),
            pl.BlockSpec((8, H1), lambda i: (0, 0)),
            pl.BlockSpec((8, H1), lambda i: (0, 0)),
        ],
        out_specs=pl.BlockSpec((2, EB, DW), lambda i: (0, i, 0)),
        out_shape=jax.ShapeDtypeStruct((2, eh, DW), _f32),
    )(xls, xrd, a_mat, bl, br)


def _edge2_body(xls_ref, xrd_ref, a_ref, bl_ref, br_ref, out_ref):
    bound = jnp.max(bl_ref[...]) + jnp.max(br_ref[...])
    z = xls_ref[...] + xrd_ref[...]
    m = jnp.maximum(z, 0.2 * z)
    e = jnp.dot(m, a_ref[...], preferred_element_type=_f32)
    w = jnp.exp(e - bound)
    out_ref[...] = jnp.concatenate(
        [w * xls_ref[:, :C2], w, jnp.zeros((EB, DW - C2 - 1), _f32)],
        axis=1)


def _tc_edge2(xls, xrd, a_mat, bl, br):
    eh = xls.shape[0]
    return pl.pallas_call(
        _edge2_body,
        grid=(eh // EB,),
        in_specs=[
            pl.BlockSpec((EB, DW), lambda i: (i, 0)),
            pl.BlockSpec((EB, DW), lambda i: (i, 0)),
            pl.BlockSpec((DW, 1), lambda i: (0, 0)),
            pl.BlockSpec((8, 1), lambda i: (0, 0)),
            pl.BlockSpec((8, 1), lambda i: (0, 0)),
        ],
        out_specs=pl.BlockSpec((EB, DW), lambda i: (i, 0)),
        out_shape=jax.ShapeDtypeStruct((eh, DW), _f32),
    )(xls, xrd, a_mat, bl, br)


def _combine1_body(p0a_ref, p1a_ref, p0b_ref, p1b_ref, st_ref, b_ref,
                   h_ref, ps_ref, pq_ref):
    p0 = p0a_ref[...] + p0b_ref[...]
    p1 = p1a_ref[...] + p1b_ref[...]
    num = jnp.concatenate([p0[:, 0:2 * C1], p1[:, 0:2 * C1]], axis=1)
    den4 = jnp.concatenate([p0[:, 2 * C1:2 * C1 + 2],
                            p1[:, 2 * C1:2 * C1 + 2]], axis=1)
    den = jnp.dot(den4, st_ref[...], preferred_element_type=_f32) + 1e-16
    h = jnp.maximum(num / den + b_ref[...], 0.0)
    rows = (pl.program_id(0) * RB
            + lax.broadcasted_iota(jnp.int32, (RB, 1), 0))
    h = jnp.where(rows < N, h, 0.0)
    h_ref[...] = h

    @pl.when(pl.program_id(0) == 0)
    def _():
        ps_ref[...] = jnp.zeros(ps_ref.shape, _f32)
        pq_ref[...] = jnp.zeros(pq_ref.shape, _f32)

    ps_ref[...] = ps_ref[...] + jnp.sum(h, axis=0, keepdims=True)
    pq_ref[...] = pq_ref[...] + jnp.sum(h * h, axis=0, keepdims=True)


def _tc_combine1(p0a, p1a, p0b, p1b, st_mat, b1):
    return pl.pallas_call(
        _combine1_body,
        grid=(NRB,),
        in_specs=[
            pl.BlockSpec((RB, DW), lambda i: (i, 0)),
            pl.BlockSpec((RB, DW), lambda i: (i, 0)),
            pl.BlockSpec((RB, DW), lambda i: (i, 0)),
            pl.BlockSpec((RB, DW), lambda i: (i, 0)),
            pl.BlockSpec((H1, D1), lambda i: (0, 0)),
            pl.BlockSpec((1, D1), lambda i: (0, 0)),
        ],
        out_specs=[
            pl.BlockSpec((RB, D1), lambda i: (i, 0)),
            pl.BlockSpec((8, D1), lambda i: (0, 0)),
            pl.BlockSpec((8, D1), lambda i: (0, 0)),
        ],
        out_shape=[
            jax.ShapeDtypeStruct((NP, D1), _f32),
            jax.ShapeDtypeStruct((8, D1), _f32),
            jax.ShapeDtypeStruct((8, D1), _f32),
        ],
    )(p0a, p1a, p0b, p1b, st_mat, b1)


def _bn_mm2_body(h_ref, ps_ref, pq_ref, g_ref, bt_ref, wl_ref, wr_ref,
                 aabs_ref, xl_ref, xr_ref, bl_ref, br_ref):
    mu = ps_ref[0:1, :] / N
    var = pq_ref[0:1, :] / N - mu * mu
    hn = (h_ref[...] - mu) / jnp.sqrt(var + EPS) * g_ref[...] + bt_ref[...]
    rows = lax.broadcasted_iota(jnp.int32, (NP, 1), 0)
    hn = jnp.where(rows < N, hn, 0.0)
    xl = jnp.dot(hn, wl_ref[...], preferred_element_type=_f32)
    xr = jnp.dot(hn, wr_ref[...], preferred_element_type=_f32)
    xl_ref[...] = xl
    xr_ref[...] = xr
    aabs = aabs_ref[...]
    zero8 = jnp.zeros((8, 1), _f32)
    bl_ref[...] = zero8 + jnp.max(
        jnp.dot(jnp.abs(xl), aabs, preferred_element_type=_f32))
    br_ref[...] = zero8 + jnp.max(
        jnp.dot(jnp.abs(xr), aabs, preferred_element_type=_f32))


def _tc_bn_mm2(h, ps, pq, g, bt, wl, wr, aabs):
    return pl.pallas_call(
        _bn_mm2_body,
        out_shape=[
            jax.ShapeDtypeStruct((NP, DW), _f32),
            jax.ShapeDtypeStruct((NP, DW), _f32),
            jax.ShapeDtypeStruct((8, 1), _f32),
            jax.ShapeDtypeStruct((8, 1), _f32),
        ],
    )(h, ps, pq, g, bt, wl, wr, aabs)


def _final_body(p0a_ref, p1a_ref, p0b_ref, p1b_ref, b_ref, g_ref, bt_ref,
                batch_ref, wlin_ref, blin_ref, out_ref):
    acc = (p0a_ref[...] + p1a_ref[...] + p0b_ref[...] + p1b_ref[...])
    den = acc[:, C2:C2 + 1] + 1e-16
    h = jnp.maximum(acc[:, :C2] / den + b_ref[...], 0.0)
    rows = lax.broadcasted_iota(jnp.int32, (NP, 1), 0)
    h = jnp.where(rows < N, h, 0.0)
    mu = jnp.sum(h, axis=0, keepdims=True) / N
    var = jnp.sum(h * h, axis=0, keepdims=True) / N - mu * mu
    hn = (h - mu) / jnp.sqrt(var + EPS) * g_ref[...] + bt_ref[...]
    hn = jnp.where(rows < N, hn, 0.0)
    batch = batch_ref[...]

    def pool_body(g, pool):
        bb = batch == g
        s = jnp.sum(jnp.where(bb, hn, 0.0), axis=0, keepdims=True)
        cnt = jnp.sum(jnp.where(bb, 1.0, 0.0))
        mx = jnp.max(jnp.where(bb, hn, -jnp.inf), axis=0, keepdims=True)
        mean = s / jnp.maximum(cnt, 1.0)
        mx = jnp.where(cnt > 0, mx, 0.0)
        row = jnp.concatenate([s, mean, mx], axis=1)
        sel = (lax.broadcasted_iota(jnp.int32, (NG, 1), 0) == g
               ).astype(_f32)
        return pool + sel * row

    pool = lax.fori_loop(0, NG, pool_body, jnp.zeros((NG, 3 * C2), _f32))
    out_ref[...] = (jnp.dot(pool, wlin_ref[...], preferred_element_type=_f32)
                    + blin_ref[...])


def _tc_final(p0a, p1a, p0b, p1b, b2, g2, bt2, batch_p, wlin, blin):
    return pl.pallas_call(
        _final_body,
        out_shape=jax.ShapeDtypeStruct((NG, 16), _f32),
    )(p0a, p1a, p0b, p1b, b2, g2, bt2, batch_p, wlin, blin)


def kernel(x, edge_index, batch, Wl1, Wr1, att1, b1, g1, bt1,
           Wl2, Wr2, att2, b2, g2, bt2, Wlin, blin):
    idt = edge_index.dtype
    sl = jnp.arange(N, dtype=idt)
    npad = EP - (edge_index.shape[1] + N)
    src = jnp.concatenate(
        [edge_index[0], sl, jnp.full((npad,), N, idt)]).astype(jnp.int32)
    dst = jnp.concatenate(
        [edge_index[1], sl, jnp.full((npad,), N, idt)]).astype(jnp.int32)
    x_p = jnp.pad(x, ((0, NP - N), (0, 0)))
    batch_p = jnp.pad(batch, (0, NP - N), constant_values=NG)
    batch_p = batch_p.reshape(NP, 1).astype(jnp.int32)

    ch = jnp.arange(D1)
    A1 = jnp.zeros((D1, H1), _f32).at[ch, ch // C1].set(
        att1[ch // C1, ch % C1])
    ST1 = (ch[None, :] // C1 == jnp.arange(H1)[:, None]).astype(_f32)
    A2 = jnp.pad(att2.reshape(C2, 1), ((0, DW - C2), (0, 0)))
    Wl2p = jnp.pad(Wl2, ((0, 0), (0, DW - C2)))
    Wr2p = jnp.pad(Wr2, ((0, 0), (0, DW - C2)))

    CA = 1312
    CB = NCHUNK * NW - CA
    xl1, xr1, bl1, br1 = _tc_mm2(x_p, Wl1, Wr1, jnp.abs(A1))
    xlsA, xrdA = _sc_gather2(xl1, xr1, src, dst, 0, CA // NW)
    xlsB, xrdB = _sc_gather2(xl1, xr1, src, dst, CA, CB // NW)
    stA = _tc_edge1(xlsA, xrdA, A1, bl1, br1)
    stB = _tc_edge1(xlsB, xrdB, A1, bl1, br1)
    pA = _sc_scatter(stA, dst, True, 0, CA)
    pB = _sc_scatter(stB, dst, True, CA, CB)
    h1, ps1, pq1 = _tc_combine1(pA[0], pA[1], pB[0], pB[1], ST1,
                                b1.reshape(1, D1))
    xl2, xr2, bl2, br2 = _tc_bn_mm2(h1, ps1, pq1, g1.reshape(1, D1),
                                    bt1.reshape(1, D1), Wl2p, Wr2p,
                                    jnp.abs(A2))

    xls2A, xrd2A = _sc_gather2(xl2, xr2, src, dst, 0, CA // NW)
    xls2B, xrd2B = _sc_gather2(xl2, xr2, src, dst, CA, CB // NW)
    st2A = _tc_edge2(xls2A, xrd2A, A2, bl2, br2)
    st2B = _tc_edge2(xls2B, xrd2B, A2, bl2, br2)
    p2A = _sc_scatter(st2A, dst, False, 0, CA)
    p2B = _sc_scatter(st2B, dst, False, CA, CB)

    return _tc_final(p2A[0], p2A[1], p2B[0], p2B[1], b2.reshape(1, C2),
                     g2.reshape(1, C2), bt2.reshape(1, C2),
                     batch_p, Wlin, blin.reshape(1, 16))
